# baseline (speedup 1.0000x reference)
import jax
import jax.numpy as jnp
from jax import lax
from jax.experimental import pallas as pl
from jax.experimental.pallas import tpu as pltpu
from jax.experimental.pallas import tpu_sc as plsc

_IGNORE_INDEX = 10

_NC = 2
_NS = 16
_NW = _NC * _NS
_L = 16

_ZLEN = 16000


def _make_sc_kernel(N, C):
    rows_per_w = N // _NW
    region = rows_per_w * C
    n_zero_dmas = region // _ZLEN
    assert region % _ZLEN == 0
    n_vec = rows_per_w // _L
    idx_rows = rows_per_w // 128
    assert rows_per_w % 128 == 0

    mesh = plsc.VectorSubcoreMesh(core_axis_name="c", subcore_axis_name="s")

    def body(grad_hbm, tgt_hbm, out_hbm, zbuf, tgt_v, grd_v, idx_v, val_v,
             zsem, ssem):
        wid = lax.axis_index("s") * _NC + lax.axis_index("c")
        row_base = wid * rows_per_w
        flat_base = wid * region

        zeros16 = jnp.zeros((_L,), jnp.float32)

        def zero_step(i, carry):
            base = i * (8 * _L)
            for k in range(8):
                zbuf[pl.ds(base + k * _L, _L)] = zeros16
            return carry

        lax.fori_loop(0, _ZLEN // (8 * _L), zero_step, 0)

        fills = []
        for j in range(n_zero_dmas):
            fills.append(
                pltpu.async_copy(
                    zbuf, out_hbm.at[pl.ds(flat_base + j * _ZLEN, _ZLEN)],
                    zsem))

        pltpu.sync_copy(tgt_hbm.at[pl.ds(row_base, rows_per_w)], tgt_v)
        pltpu.sync_copy(grad_hbm.at[pl.ds(row_base, rows_per_w)], grd_v)

        lane = lax.iota(jnp.int32, _L)
        for i in range(n_vec):
            t = tgt_v[pl.ds(i * _L, _L)]
            g = grd_v[pl.ds(i * _L, _L)]
            valid = t != _IGNORE_INDEX
            vals = jnp.where(valid, -g, jnp.zeros((_L,), jnp.float32))
            rows = (row_base + i * _L) + lane
            flat = rows * C + t
            r, col = i // 8, (i % 8) * _L
            idx_v[r, pl.ds(col, _L)] = flat
            val_v[r, pl.ds(col, _L)] = vals

        for f in fills:
            f.wait()

        scats = []
        for r in range(idx_rows):
            scats.append(
                pltpu.async_copy(val_v.at[r], out_hbm.at[idx_v.at[r]], ssem))
        for s in scats:
            s.wait()

    kern = pl.kernel(
        body,
        mesh=mesh,
        out_type=jax.ShapeDtypeStruct((N * C,), jnp.float32),
        scratch_types=[
            pltpu.VMEM((_ZLEN,), jnp.float32),
            pltpu.VMEM((rows_per_w,), jnp.int32),
            pltpu.VMEM((rows_per_w,), jnp.float32),
            pltpu.VMEM((idx_rows, 128), jnp.int32),
            pltpu.VMEM((idx_rows, 128), jnp.float32),
            pltpu.SemaphoreType.DMA,
            pltpu.SemaphoreType.DMA,
        ],
    )
    return kern


def kernel(grad_output, input, target, total_weight):
    N, C = input.shape
    kern = _make_sc_kernel(N, C)
    out_flat = kern(grad_output.astype(jnp.float32),
                    target.astype(jnp.int32))
    return out_flat.reshape(N, C)


# speedup vs baseline: 1.0346x; 1.0346x over previous
import jax
import jax.numpy as jnp
from jax import lax
from jax.experimental import pallas as pl
from jax.experimental.pallas import tpu as pltpu
from jax.experimental.pallas import tpu_sc as plsc

_IGNORE_INDEX = 10

_NC = 2
_NS = 16
_NW = _NC * _NS
_L = 16

_ZLEN = 16384


def _make_sc_kernel(N, C):
    CP = ((C + 127) // 128) * 128
    rows_per_w = N // _NW
    region = rows_per_w * CP
    n_zero_dmas = region // _ZLEN
    assert N % _NW == 0 and region % _ZLEN == 0 and rows_per_w % 128 == 0
    n_vec = rows_per_w // _L
    idx_rows = rows_per_w // 128

    mesh = plsc.VectorSubcoreMesh(core_axis_name="c", subcore_axis_name="s")

    def body(grad_hbm, tgt_hbm, out_hbm, zbuf, tgt_v, grd_v, idx_v, val_v,
             zsem, ssem):
        wid = lax.axis_index("s") * _NC + lax.axis_index("c")
        row_base = wid * rows_per_w
        flat_base = wid * region

        zeros16 = jnp.zeros((_L,), jnp.float32)

        def zero_step(i, carry):
            base = i * (8 * _L)
            for k in range(8):
                zbuf[pl.ds(base + k * _L, _L)] = zeros16
            return carry

        lax.fori_loop(0, _ZLEN // (8 * _L), zero_step, 0)

        fills = []
        for j in range(n_zero_dmas):
            fills.append(
                pltpu.async_copy(
                    zbuf, out_hbm.at[pl.ds(flat_base + j * _ZLEN, _ZLEN)],
                    zsem))

        pltpu.sync_copy(tgt_hbm.at[pl.ds(row_base, rows_per_w)], tgt_v)
        pltpu.sync_copy(grad_hbm.at[pl.ds(row_base, rows_per_w)], grd_v)

        lane = lax.iota(jnp.int32, _L)
        for i in range(n_vec):
            t = tgt_v[pl.ds(i * _L, _L)]
            g = grd_v[pl.ds(i * _L, _L)]
            valid = t != _IGNORE_INDEX
            vals = jnp.where(valid, -g, jnp.zeros((_L,), jnp.float32))
            rows = (row_base + i * _L) + lane
            flat = rows * CP + t
            r, col = i // 8, (i % 8) * _L
            idx_v[r, pl.ds(col, _L)] = flat
            val_v[r, pl.ds(col, _L)] = vals

        for f in fills:
            f.wait()

        scats = []
        for r in range(idx_rows):
            scats.append(
                pltpu.async_copy(val_v.at[r], out_hbm.at[idx_v.at[r]], ssem))
        for s in scats:
            s.wait()

    kern = pl.kernel(
        body,
        mesh=mesh,
        out_type=jax.ShapeDtypeStruct((N * CP,), jnp.float32),
        scratch_types=[
            pltpu.VMEM((_ZLEN,), jnp.float32),
            pltpu.VMEM((rows_per_w,), jnp.int32),
            pltpu.VMEM((rows_per_w,), jnp.float32),
            pltpu.VMEM((idx_rows, 128), jnp.int32),
            pltpu.VMEM((idx_rows, 128), jnp.float32),
            pltpu.SemaphoreType.DMA,
            pltpu.SemaphoreType.DMA,
        ],
    )
    return kern, CP


def kernel(grad_output, input, target, total_weight):
    N, C = input.shape
    kern, CP = _make_sc_kernel(N, C)
    out_flat = kern(grad_output.astype(jnp.float32),
                    target.astype(jnp.int32))
    return out_flat.reshape(N, CP)[:, :C]


# speedup vs baseline: 1.3732x; 1.3273x over previous
import jax
import jax.numpy as jnp
from jax import lax
from jax.experimental import pallas as pl
from jax.experimental.pallas import tpu as pltpu
from jax.experimental.pallas import tpu_sc as plsc

_IGNORE_INDEX = 10

_NC = 2
_NS = 16
_NW = _NC * _NS
_L = 16

_CHUNK_ROWS = 16


def _window_offsets(C):
    n_full = C // _L
    offs = [w * _L for w in range(n_full)]
    if C % _L:
        offs.append(C - _L)
    return offs


def _make_sc_kernel(N, C):
    rows_per_w = N // _NW
    n_chunks = rows_per_w // _CHUNK_ROWS
    assert N % _NW == 0 and rows_per_w % _CHUNK_ROWS == 0 and n_chunks >= 4
    offs = _window_offsets(C)
    n_win = len(offs)

    mesh = plsc.VectorSubcoreMesh(core_axis_name="c", subcore_axis_name="s")

    def body(src_hbm, win_hbm, out_hbm, buf0, buf1, src_v, win_v,
             sem0, sem1):
        worker = lax.axis_index("s") * _NC + lax.axis_index("c")
        row_base = worker * rows_per_w

        pltpu.sync_copy(src_hbm.at[pl.ds(row_base * _L, rows_per_w * _L)],
                        src_v)
        pltpu.sync_copy(win_hbm.at[pl.ds(row_base * _L, rows_per_w * _L)],
                        win_v)

        zeros16 = jnp.zeros((_L,), jnp.float32)

        def build_row(buf, r, lane_off):
            sv = src_v[pl.ds(lane_off, _L)]
            wv = win_v[pl.ds(lane_off, _L)]
            for j in range(n_win - 1, -1, -1):
                buf[r, pl.ds(offs[j], _L)] = jnp.where(
                    wv == j, sv, zeros16)

        def do_chunk(chunk, buf, sem):
            @pl.when(chunk >= 2)
            def _():
                pltpu.make_async_copy(
                    buf,
                    out_hbm.at[pl.ds(row_base + (chunk - 2) * _CHUNK_ROWS,
                                     _CHUNK_ROWS), :],
                    sem).wait()

            def row_step(r, carry):
                build_row(buf, r, (chunk * _CHUNK_ROWS + r) * _L)
                return carry

            lax.fori_loop(0, _CHUNK_ROWS, row_step, 0)
            pltpu.async_copy(
                buf,
                out_hbm.at[pl.ds(row_base + chunk * _CHUNK_ROWS,
                                 _CHUNK_ROWS), :],
                sem)

        def step(chunk, carry):
            @pl.when(chunk % 2 == 0)
            def _():
                do_chunk(chunk, buf0, sem0)

            @pl.when(chunk % 2 == 1)
            def _():
                do_chunk(chunk, buf1, sem1)

            return carry

        lax.fori_loop(0, n_chunks, step, 0)

        pltpu.make_async_copy(
            buf0,
            out_hbm.at[pl.ds(row_base + (n_chunks - 2) * _CHUNK_ROWS,
                             _CHUNK_ROWS), :],
            sem0).wait()
        pltpu.make_async_copy(
            buf1,
            out_hbm.at[pl.ds(row_base + (n_chunks - 1) * _CHUNK_ROWS,
                             _CHUNK_ROWS), :],
            sem1).wait()

    kern = pl.kernel(
        body,
        mesh=mesh,
        compiler_params=pltpu.CompilerParams(use_tc_tiling_on_sc=True),
        out_type=jax.ShapeDtypeStruct((N, C), jnp.float32),
        scratch_types=[
            pltpu.VMEM((_CHUNK_ROWS, C), jnp.float32),
            pltpu.VMEM((_CHUNK_ROWS, C), jnp.float32),
            pltpu.VMEM((rows_per_w * _L,), jnp.float32),
            pltpu.VMEM((rows_per_w * _L,), jnp.int32),
            pltpu.SemaphoreType.DMA,
            pltpu.SemaphoreType.DMA,
        ],
    )
    return kern


def kernel(grad_output, input, target, total_weight):
    N, C = input.shape
    t = target.astype(jnp.int32)
    g = grad_output.astype(jnp.float32)
    vals = jnp.where(t != _IGNORE_INDEX, -g, jnp.zeros_like(g))
    win = t // _L
    n_win = len(_window_offsets(C))
    win = jnp.minimum(win, n_win - 1)
    woff = jnp.where(win == n_win - 1, C - _L, win * _L)
    pos = t - woff
    lanes = jnp.arange(_L, dtype=jnp.int32)
    srcmat = jnp.where(pos[:, None] == lanes[None, :], vals[:, None],
                       jnp.float32(0)).reshape(N * _L)
    winmat = jnp.broadcast_to(win[:, None], (N, _L)).reshape(N * _L)
    kern = _make_sc_kernel(N, C)
    return kern(srcmat, winmat)
